# baseline (speedup 1.0000x reference)
import functools

import jax
import jax.numpy as jnp
from jax import lax
from jax.experimental import pallas as pl
from jax.experimental.pallas import tpu as pltpu
from jax.experimental.pallas import tpu_sc as plsc

N_NODES = 10000
PART_EMB = 64
NODE_FEAT = 128
HIDDEN = 128

_NC = 2
_NS = 16
_NW = _NC * _NS
B_PAD = 10240
B_PER_W = B_PAD // _NW
CHUNK = 80
N_CHUNKS = B_PER_W // CHUNK


@functools.partial(
    pl.kernel,
    mesh=plsc.VectorSubcoreMesh(core_axis_name="c", subcore_axis_name="s"),
    out_type=jax.ShapeDtypeStruct((B_PAD, PART_EMB), jnp.float32),
    scratch_types=[
        pltpu.VMEM((B_PER_W,), jnp.int32),
        pltpu.VMEM((B_PER_W, PART_EMB), jnp.float32),
        pltpu.SemaphoreType.DMA,
    ],
    compiler_params=pltpu.CompilerParams(use_tc_tiling_on_sc=False),
)
def _sc_gather(table_hbm, idx_hbm, out_hbm, idx_v, rows_v, gsem):
    wid = lax.axis_index("s") * _NC + lax.axis_index("c")
    base = wid * B_PER_W
    pltpu.sync_copy(idx_hbm.at[pl.ds(base, B_PER_W)], idx_v)
    copies = []
    for ci in range(N_CHUNKS):
        copies.append(
            pltpu.async_copy(
                table_hbm.at[idx_v.at[pl.ds(ci * CHUNK, CHUNK)]],
                rows_v.at[pl.ds(ci * CHUNK, CHUNK)],
                gsem,
            )
        )
    for c in copies:
        c.wait()
    pltpu.sync_copy(rows_v, out_hbm.at[pl.ds(base, B_PER_W)])


ROWS_BLK = 2000
_GRID = N_NODES // ROWS_BLK


def _tc_dense_body(x_ref, w2_ref, b_ref, out_ref):
    out_ref[...] = (
        jnp.dot(x_ref[...], w2_ref[...], preferred_element_type=jnp.float32)
        + b_ref[...]
    )


_tc_dense = pl.pallas_call(
    _tc_dense_body,
    grid=(_GRID,),
    in_specs=[
        pl.BlockSpec((ROWS_BLK, NODE_FEAT), lambda i: (i, 0)),
        pl.BlockSpec((NODE_FEAT, HIDDEN), lambda i: (0, 0)),
        pl.BlockSpec((1, HIDDEN), lambda i: (0, 0)),
    ],
    out_specs=pl.BlockSpec((ROWS_BLK, HIDDEN), lambda i: (i, 0)),
    out_shape=jax.ShapeDtypeStruct((N_NODES, HIDDEN), jnp.float32),
)


def _tc_emb_body(pe_ref, acc_ref, w1_ref, out_ref):
    out_ref[...] = acc_ref[...] + jnp.dot(
        pe_ref[...], w1_ref[...], preferred_element_type=jnp.float32
    )


_tc_emb = pl.pallas_call(
    _tc_emb_body,
    grid=(_GRID,),
    in_specs=[
        pl.BlockSpec((ROWS_BLK, PART_EMB), lambda i: (i, 0)),
        pl.BlockSpec((ROWS_BLK, HIDDEN), lambda i: (i, 0)),
        pl.BlockSpec((PART_EMB, HIDDEN), lambda i: (0, 0)),
    ],
    out_specs=pl.BlockSpec((ROWS_BLK, HIDDEN), lambda i: (i, 0)),
    out_shape=jax.ShapeDtypeStruct((N_NODES, HIDDEN), jnp.float32),
)


def kernel(x, edge_index, part_ids, embeddings, W, b):
    del edge_index
    pad = jnp.arange(B_PAD - N_NODES, dtype=jnp.int32)
    idx = jnp.concatenate([part_ids.astype(jnp.int32), pad])
    pe = _sc_gather(embeddings, idx)
    Wt = W.T
    w1 = Wt[:PART_EMB]
    w2 = Wt[PART_EMB:]
    acc = _tc_dense(x, w2, b.reshape(1, HIDDEN))
    return _tc_emb(pe, acc, w1)


# speedup vs baseline: 1.5249x; 1.5249x over previous
import functools

import jax
import jax.numpy as jnp
from jax import lax
from jax.experimental import pallas as pl
from jax.experimental.pallas import tpu as pltpu
from jax.experimental.pallas import tpu_sc as plsc

N_NODES = 10000
PART_EMB = 64
NODE_FEAT = 128
HIDDEN = 128

_NC = 2
_NS = 16
_NW = _NC * _NS
B_PAD = 10240
B_PER_W = B_PAD // _NW
CHUNK = 80
N_CHUNKS = B_PER_W // CHUNK


@functools.partial(
    pl.kernel,
    mesh=plsc.VectorSubcoreMesh(core_axis_name="c", subcore_axis_name="s"),
    out_type=jax.ShapeDtypeStruct((B_PAD, PART_EMB), jnp.float32),
    scratch_types=[
        pltpu.VMEM((B_PER_W,), jnp.int32),
        pltpu.VMEM((B_PER_W, PART_EMB), jnp.float32),
        pltpu.SemaphoreType.DMA,
    ],
)
def _sc_gather(table_hbm, idx_hbm, out_hbm, idx_v, rows_v, gsem):
    wid = lax.axis_index("s") * _NC + lax.axis_index("c")
    base = wid * B_PER_W
    pltpu.sync_copy(idx_hbm.at[pl.ds(base, B_PER_W)], idx_v)

    def issue_group(g, _):
        v = idx_v[pl.ds(g * 16, 16)]
        for j in range(16):
            pltpu.async_copy(
                table_hbm.at[pl.ds(v[j], 1)],
                rows_v.at[pl.ds(g * 16 + j, 1)],
                gsem,
            )
        return ()

    lax.fori_loop(0, B_PER_W // 16, issue_group, ())

    pltpu.make_async_copy(
        table_hbm.at[pl.ds(0, B_PER_W)], rows_v, gsem
    ).wait()
    pltpu.sync_copy(rows_v, out_hbm.at[pl.ds(base, B_PER_W)])


ROWS_BLK = 2000
_GRID = N_NODES // ROWS_BLK


def _tc_body(pe_ref, x_ref, w1_ref, w2_ref, b_ref, out_ref):
    out_ref[...] = (
        jnp.dot(pe_ref[...], w1_ref[...], preferred_element_type=jnp.float32)
        + jnp.dot(x_ref[...], w2_ref[...], preferred_element_type=jnp.float32)
        + b_ref[...]
    )


_tc_encode = pl.pallas_call(
    _tc_body,
    grid=(_GRID,),
    in_specs=[
        pl.BlockSpec((ROWS_BLK, PART_EMB), lambda i: (i, 0)),
        pl.BlockSpec((ROWS_BLK, NODE_FEAT), lambda i: (i, 0)),
        pl.BlockSpec((PART_EMB, HIDDEN), lambda i: (0, 0)),
        pl.BlockSpec((NODE_FEAT, HIDDEN), lambda i: (0, 0)),
        pl.BlockSpec((1, HIDDEN), lambda i: (0, 0)),
    ],
    out_specs=pl.BlockSpec((ROWS_BLK, HIDDEN), lambda i: (i, 0)),
    out_shape=jax.ShapeDtypeStruct((N_NODES, HIDDEN), jnp.float32),
)


def kernel(x, edge_index, part_ids, embeddings, W, b):
    del edge_index
    pad = jnp.arange(B_PAD - N_NODES, dtype=jnp.int32)
    idx = jnp.concatenate([part_ids.astype(jnp.int32), pad])
    pe = _sc_gather(embeddings, idx)
    Wt = W.T
    w1 = Wt[:PART_EMB]
    w2 = Wt[PART_EMB:]
    return _tc_encode(pe, x, w1, w2, b.reshape(1, HIDDEN))
